# baseline (speedup 1.0000x reference)
import functools

import jax
import jax.numpy as jnp
from jax import lax
from jax.experimental import pallas as pl
from jax.experimental.pallas import tpu as pltpu
from jax.experimental.pallas import tpu_sc as plsc

C = 1000
B = 4096
NC = 2
NS = 16
NW = NC * NS
RW = B // NW
GR = 16
NG = RW // GR


def _sc_body(sv_hbm, lab_hbm, coarse_hbm, fine_hbm,
             out_hbm, m2_hbm, se_hbm, svl_hbm,
             svbuf, fbuf, idxbuf, labbuf, coarsebuf,
             m2buf, sebuf, svlbuf, sem):
  cid = lax.axis_index("c")
  sid = lax.axis_index("s")
  wid = sid * NC + cid
  base = wid * RW
  riota = lax.iota(jnp.int32, 16)

  pltpu.sync_copy(coarse_hbm, coarsebuf)
  pltpu.sync_copy(lab_hbm.at[pl.ds(base, RW)], labbuf)

  def group(g, _):
    rb = base + g * GR
    pltpu.sync_copy(sv_hbm.at[pl.ds(rb, GR), :], svbuf)

    def p1(j, carry):
      mx, am = carry
      jv = jnp.full((16,), j, jnp.int32)
      v = plsc.load_gather(svbuf, [riota, jv])
      p = v > mx
      return jnp.where(p, v, mx), jnp.where(p, jv, am)

    neg_inf = jnp.full((16,), -jnp.inf, jnp.float32)
    _, am = lax.fori_loop(0, C, p1, (neg_inf, jnp.zeros((16,), jnp.int32)))
    idxbuf[...] = am

    pltpu.async_copy(fine_hbm.at[idxbuf], fbuf, sem).wait()
    cv = plsc.load_gather(coarsebuf, [am])
    labv = labbuf[pl.ds(g * GR, GR)]

    def p2(j, carry):
      m2, svl = carry
      jv = jnp.full((16,), j, jnp.int32)
      v = plsc.load_gather(svbuf, [riota, jv])
      f = plsc.load_gather(fbuf, [riota, jv])
      s = v / (cv * f)
      plsc.store_scatter(svbuf, [riota, jv], s)
      return jnp.maximum(m2, s), svl + jnp.where(labv == jv, s, 0.0)

    m2, svl = lax.fori_loop(0, C, p2, (neg_inf, jnp.zeros((16,), jnp.float32)))

    def p2b(j, acc):
      jv = jnp.full((16,), j, jnp.int32)
      s = plsc.load_gather(svbuf, [riota, jv])
      return acc + jnp.exp(s - m2)

    se = lax.fori_loop(0, C, p2b, jnp.zeros((16,), jnp.float32))

    m2buf[pl.ds(g * GR, GR)] = m2
    sebuf[pl.ds(g * GR, GR)] = se
    svlbuf[pl.ds(g * GR, GR)] = svl
    pltpu.sync_copy(svbuf, out_hbm.at[pl.ds(rb, GR), :])
    return 0

  lax.fori_loop(0, NG, group, 0)
  pltpu.sync_copy(m2buf, m2_hbm.at[pl.ds(base, RW)])
  pltpu.sync_copy(sebuf, se_hbm.at[pl.ds(base, RW)])
  pltpu.sync_copy(svlbuf, svl_hbm.at[pl.ds(base, RW)])


def _tc_loss_body(m2_ref, se_ref, svl_ref, fine_ref, out_ref):
  nll = jnp.sum(m2_ref[...] + jnp.log(se_ref[...]) - svl_ref[...]) / B
  reg = jnp.sum(jnp.abs(fine_ref[...] - 1.0)) / (C * C)
  out_ref[...] = jnp.full((1, 1), nll + reg, jnp.float32)


def kernel(Simple_vector, label_list, coarse_scaling_vector, fine_scaling_matrix):
  sc = pl.kernel(
      _sc_body,
      out_type=(
          jax.ShapeDtypeStruct((B, C), jnp.float32),
          jax.ShapeDtypeStruct((B,), jnp.float32),
          jax.ShapeDtypeStruct((B,), jnp.float32),
          jax.ShapeDtypeStruct((B,), jnp.float32),
      ),
      mesh=plsc.VectorSubcoreMesh(core_axis_name="c", subcore_axis_name="s"),
      compiler_params=pltpu.CompilerParams(use_tc_tiling_on_sc=False,
                                           needs_layout_passes=False),
      scratch_types=[
          pltpu.VMEM((GR, C), jnp.float32),
          pltpu.VMEM((GR, C), jnp.float32),
          pltpu.VMEM((GR,), jnp.int32),
          pltpu.VMEM((RW,), jnp.int32),
          pltpu.VMEM((C,), jnp.float32),
          pltpu.VMEM((RW,), jnp.float32),
          pltpu.VMEM((RW,), jnp.float32),
          pltpu.VMEM((RW,), jnp.float32),
          pltpu.SemaphoreType.DMA,
      ],
  )
  sv, m2, se, svl = sc(Simple_vector, label_list,
                       coarse_scaling_vector, fine_scaling_matrix)

  loss2 = pl.pallas_call(
      _tc_loss_body,
      out_shape=jax.ShapeDtypeStruct((1, 1), jnp.float32),
  )(m2.reshape(32, 128), se.reshape(32, 128), svl.reshape(32, 128),
    fine_scaling_matrix)
  loss = loss2[0, 0]
  return (sv, loss, jnp.zeros((), jnp.float32))


# speedup vs baseline: 1.2408x; 1.2408x over previous
import functools

import jax
import jax.numpy as jnp
from jax import lax
from jax.experimental import pallas as pl
from jax.experimental.pallas import tpu as pltpu
from jax.experimental.pallas import tpu_sc as plsc

C = 1000
B = 4096
NC = 2
NS = 16
NW = NC * NS
RW = B // NW
GR = 16
NG = RW // GR


def _sc_body(sv_hbm, lab_hbm, coarse_hbm, fine_hbm,
             out_hbm, m2_hbm, se_hbm, svl_hbm,
             svbuf, fbuf, idxbuf, labbuf, coarsebuf,
             m2buf, sebuf, svlbuf, sem):
  cid = lax.axis_index("c")
  sid = lax.axis_index("s")
  wid = sid * NC + cid
  base = wid * RW
  riota = lax.iota(jnp.int32, 16)

  pltpu.sync_copy(coarse_hbm, coarsebuf)
  pltpu.sync_copy(lab_hbm.at[pl.ds(base, RW)], labbuf)

  def merge_first(a, b):
    va, ia = a
    vb, ib = b
    p = va >= vb
    return jnp.where(p, va, vb), jnp.where(p, ia, ib)

  U = 8

  def group(g, _):
    rb = base + g * GR
    pltpu.sync_copy(sv_hbm.at[pl.ds(rb, GR), :], svbuf)

    def p1(jb, carry):
      j0 = jb * U
      cand = []
      for u in range(U):
        jv = jnp.full((16,), j0 + u, jnp.int32)
        cand.append((plsc.load_gather(svbuf, [riota, jv]), jv))
      while len(cand) > 1:
        cand = [merge_first(cand[i], cand[i + 1])
                for i in range(0, len(cand), 2)]
      return merge_first(carry, cand[0])

    neg_inf = jnp.full((16,), -jnp.inf, jnp.float32)
    _, am = lax.fori_loop(0, C // U, p1,
                          (neg_inf, jnp.zeros((16,), jnp.int32)))
    idxbuf[...] = am

    pltpu.async_copy(fine_hbm.at[idxbuf], fbuf, sem).wait()
    cv = plsc.load_gather(coarsebuf, [am])
    labv = labbuf[pl.ds(g * GR, GR)]

    def p2(jb, m2):
      j0 = jb * U
      svals = []
      for u in range(U):
        jv = jnp.full((16,), j0 + u, jnp.int32)
        v = plsc.load_gather(svbuf, [riota, jv])
        f = plsc.load_gather(fbuf, [riota, jv])
        s = v / (cv * f)
        plsc.store_scatter(svbuf, [riota, jv], s)
        svals.append(s)
      while len(svals) > 1:
        svals = [jnp.maximum(svals[i], svals[i + 1])
                 for i in range(0, len(svals), 2)]
      return jnp.maximum(m2, svals[0])

    m2 = lax.fori_loop(0, C // U, p2, neg_inf)
    svl = plsc.load_gather(svbuf, [riota, labv])

    def p2b(jb, acc):
      j0 = jb * U
      evals = []
      for u in range(U):
        jv = jnp.full((16,), j0 + u, jnp.int32)
        s = plsc.load_gather(svbuf, [riota, jv])
        evals.append(jnp.exp(s - m2))
      while len(evals) > 1:
        evals = [evals[i] + evals[i + 1] for i in range(0, len(evals), 2)]
      return acc + evals[0]

    se = lax.fori_loop(0, C // U, p2b, jnp.zeros((16,), jnp.float32))

    m2buf[pl.ds(g * GR, GR)] = m2
    sebuf[pl.ds(g * GR, GR)] = se
    svlbuf[pl.ds(g * GR, GR)] = svl
    pltpu.sync_copy(svbuf, out_hbm.at[pl.ds(rb, GR), :])
    return 0

  lax.fori_loop(0, NG, group, 0)
  pltpu.sync_copy(m2buf, m2_hbm.at[pl.ds(base, RW)])
  pltpu.sync_copy(sebuf, se_hbm.at[pl.ds(base, RW)])
  pltpu.sync_copy(svlbuf, svl_hbm.at[pl.ds(base, RW)])


def _tc_loss_body(m2_ref, se_ref, svl_ref, fine_ref, out_ref):
  nll = jnp.sum(m2_ref[...] + jnp.log(se_ref[...]) - svl_ref[...]) / B
  reg = jnp.sum(jnp.abs(fine_ref[...] - 1.0)) / (C * C)
  out_ref[...] = jnp.full((1, 1), nll + reg, jnp.float32)


def kernel(Simple_vector, label_list, coarse_scaling_vector, fine_scaling_matrix):
  sc = pl.kernel(
      _sc_body,
      out_type=(
          jax.ShapeDtypeStruct((B, C), jnp.float32),
          jax.ShapeDtypeStruct((B,), jnp.float32),
          jax.ShapeDtypeStruct((B,), jnp.float32),
          jax.ShapeDtypeStruct((B,), jnp.float32),
      ),
      mesh=plsc.VectorSubcoreMesh(core_axis_name="c", subcore_axis_name="s"),
      compiler_params=pltpu.CompilerParams(use_tc_tiling_on_sc=False,
                                           needs_layout_passes=False),
      scratch_types=[
          pltpu.VMEM((GR, C), jnp.float32),
          pltpu.VMEM((GR, C), jnp.float32),
          pltpu.VMEM((GR,), jnp.int32),
          pltpu.VMEM((RW,), jnp.int32),
          pltpu.VMEM((C,), jnp.float32),
          pltpu.VMEM((RW,), jnp.float32),
          pltpu.VMEM((RW,), jnp.float32),
          pltpu.VMEM((RW,), jnp.float32),
          pltpu.SemaphoreType.DMA,
      ],
  )
  sv, m2, se, svl = sc(Simple_vector, label_list,
                       coarse_scaling_vector, fine_scaling_matrix)

  loss2 = pl.pallas_call(
      _tc_loss_body,
      out_shape=jax.ShapeDtypeStruct((1, 1), jnp.float32),
  )(m2.reshape(32, 128), se.reshape(32, 128), svl.reshape(32, 128),
    fine_scaling_matrix)
  loss = loss2[0, 0]
  return (sv, loss, jnp.zeros((), jnp.float32))


# speedup vs baseline: 1.2699x; 1.0235x over previous
import functools

import jax
import jax.numpy as jnp
from jax import lax
from jax.experimental import pallas as pl
from jax.experimental.pallas import tpu as pltpu
from jax.experimental.pallas import tpu_sc as plsc

C = 1000
B = 4096
NC = 2
NS = 16
NW = NC * NS
RW = B // NW
GR = 16
NG = RW // GR


def _sc_body(sv_hbm, lab_hbm, coarse_hbm, fine_hbm,
             out_hbm, se_hbm, svl_hbm,
             svbuf, fbuf, idxbuf, labbuf, coarsebuf,
             sebuf, svlbuf, sem):
  cid = lax.axis_index("c")
  sid = lax.axis_index("s")
  wid = sid * NC + cid
  base = wid * RW
  riota = lax.iota(jnp.int32, 16)

  pltpu.sync_copy(coarse_hbm, coarsebuf)
  pltpu.sync_copy(lab_hbm.at[pl.ds(base, RW)], labbuf)

  U = 8
  neg_inf = jnp.full((16,), -jnp.inf, jnp.float32)

  def merge_am(a, b):
    va, ia = a
    vb, ib = b
    p = (va > vb) | ((va == vb) & (ia < ib))
    return jnp.where(p, va, vb), jnp.where(p, ia, ib)

  def tree(xs, f):
    while len(xs) > 1:
      xs = [f(xs[i], xs[i + 1]) for i in range(0, len(xs), 2)]
    return xs[0]

  def group(g, _):
    rb = base + g * GR
    pltpu.sync_copy(sv_hbm.at[pl.ds(rb, GR), :], svbuf)

    def p1(jb, carry):
      jv = carry[0]
      accs = list(carry[1])
      out = []
      for u in range(U):
        jvu = jv + u
        v = plsc.load_gather(svbuf, [riota, jvu])
        mx, am = accs[u]
        p = v > mx
        out.append((jnp.where(p, v, mx), jnp.where(p, jvu, am)))
      return (jv + U, tuple(out))

    acc0 = tuple((neg_inf, jnp.zeros((16,), jnp.int32)) for _ in range(U))
    _, accs = lax.fori_loop(0, C // U, p1,
                            (jnp.zeros((16,), jnp.int32), acc0))
    _, am = tree(list(accs), merge_am)
    idxbuf[...] = am

    pltpu.async_copy(fine_hbm.at[idxbuf], fbuf, sem).wait()
    cv = plsc.load_gather(coarsebuf, [am])
    labv = labbuf[pl.ds(g * GR, GR)]

    def p2(jb, carry):
      jv = carry[0]
      sums = list(carry[1])
      out = []
      for u in range(U):
        jvu = jv + u
        v = plsc.load_gather(svbuf, [riota, jvu])
        f = plsc.load_gather(fbuf, [riota, jvu])
        s = v / (cv * f)
        plsc.store_scatter(svbuf, [riota, jvu], s)
        out.append(sums[u] + jnp.exp(s))
      return (jv + U, tuple(out))

    zero = jnp.zeros((16,), jnp.float32)
    _, sums = lax.fori_loop(0, C // U, p2,
                            (jnp.zeros((16,), jnp.int32),
                             tuple(zero for _ in range(U))))
    se = tree(list(sums), jnp.add)
    svl = plsc.load_gather(svbuf, [riota, labv])

    sebuf[pl.ds(g * GR, GR)] = se
    svlbuf[pl.ds(g * GR, GR)] = svl
    pltpu.sync_copy(svbuf, out_hbm.at[pl.ds(rb, GR), :])
    return 0

  lax.fori_loop(0, NG, group, 0)
  pltpu.sync_copy(sebuf, se_hbm.at[pl.ds(base, RW)])
  pltpu.sync_copy(svlbuf, svl_hbm.at[pl.ds(base, RW)])


def _tc_loss_body(se_ref, svl_ref, fine_ref, out_ref):
  nll = jnp.sum(jnp.log(se_ref[...]) - svl_ref[...]) / B
  reg = jnp.sum(jnp.abs(fine_ref[...] - 1.0)) / (C * C)
  out_ref[...] = jnp.full((1, 1), nll + reg, jnp.float32)


def kernel(Simple_vector, label_list, coarse_scaling_vector, fine_scaling_matrix):
  sc = pl.kernel(
      _sc_body,
      out_type=(
          jax.ShapeDtypeStruct((B, C), jnp.float32),
          jax.ShapeDtypeStruct((B,), jnp.float32),
          jax.ShapeDtypeStruct((B,), jnp.float32),
      ),
      mesh=plsc.VectorSubcoreMesh(core_axis_name="c", subcore_axis_name="s"),
      compiler_params=pltpu.CompilerParams(use_tc_tiling_on_sc=False,
                                           needs_layout_passes=False),
      scratch_types=[
          pltpu.VMEM((GR, C), jnp.float32),
          pltpu.VMEM((GR, C), jnp.float32),
          pltpu.VMEM((GR,), jnp.int32),
          pltpu.VMEM((RW,), jnp.int32),
          pltpu.VMEM((C,), jnp.float32),
          pltpu.VMEM((RW,), jnp.float32),
          pltpu.VMEM((RW,), jnp.float32),
          pltpu.SemaphoreType.DMA,
      ],
  )
  sv, se, svl = sc(Simple_vector, label_list,
                       coarse_scaling_vector, fine_scaling_matrix)

  loss2 = pl.pallas_call(
      _tc_loss_body,
      out_shape=jax.ShapeDtypeStruct((1, 1), jnp.float32),
  )(se.reshape(32, 128), svl.reshape(32, 128),
    fine_scaling_matrix)
  loss = loss2[0, 0]
  return (sv, loss, jnp.zeros((), jnp.float32))


# speedup vs baseline: 1.3321x; 1.0490x over previous
import jax
import jax.numpy as jnp
from jax import lax
from jax.experimental import pallas as pl
from jax.experimental.pallas import tpu as pltpu
from jax.experimental.pallas import tpu_sc as plsc

C = 1000
B = 4096
NC = 2
NS = 16
NW = NC * NS
RW = B // NW
GR = 16
NG = RW // GR
U = 8


def _sc_body(sv_hbm, lab_hbm, coarse_hbm, fine_hbm,
             out_hbm, se_hbm, svl_hbm,
             svbufs, fbufs, idxbufs, labbuf, coarsebuf,
             sebuf, svlbuf, insems, fsems, outsems):
  cid = lax.axis_index("c")
  sid = lax.axis_index("s")
  wid = sid * NC + cid
  base = wid * RW
  riota = lax.iota(jnp.int32, 16)
  zero = jnp.zeros((16,), jnp.float32)
  neg_inf = jnp.full((16,), -jnp.inf, jnp.float32)

  pltpu.sync_copy(coarse_hbm, coarsebuf)
  pltpu.sync_copy(lab_hbm.at[pl.ds(base, RW)], labbuf)

  def merge_am(a, b):
    va, ia = a
    vb, ib = b
    p = (va > vb) | ((va == vb) & (ia < ib))
    return jnp.where(p, va, vb), jnp.where(p, ia, ib)

  def tree(xs, f):
    while len(xs) > 1:
      xs = [f(xs[i], xs[i + 1]) for i in range(0, len(xs), 2)]
    return xs[0]

  def start_in(g):
    return pltpu.async_copy(
        sv_hbm.at[pl.ds(base + g * GR, GR), :], svbufs[g % 3],
        insems[g % 3])

  def p1(g):
    svbuf = svbufs[g % 3]

    def body(jb, carry):
      jv = carry[0]
      accs = list(carry[1])
      out = []
      for u in range(U):
        jvu = jv + u
        v = plsc.load_gather(svbuf, [riota, jvu])
        mx, am = accs[u]
        p = v > mx
        out.append((jnp.where(p, v, mx), jnp.where(p, jvu, am)))
      return (jv + U, tuple(out))

    acc0 = tuple((neg_inf, jnp.zeros((16,), jnp.int32)) for _ in range(U))
    _, accs = lax.fori_loop(0, C // U, body,
                            (jnp.zeros((16,), jnp.int32), acc0))
    _, am = tree(list(accs), merge_am)
    idxbufs[g % 2][...] = am
    return am

  def start_fine(g):
    return pltpu.async_copy(fine_hbm.at[idxbufs[g % 2]], fbufs[g % 2],
                            fsems[g % 2])

  def p2(g, am):
    svbuf = svbufs[g % 3]
    fbuf = fbufs[g % 2]
    cv = plsc.load_gather(coarsebuf, [am])

    def body(jb, carry):
      jv = carry[0]
      sums = list(carry[1])
      out = []
      for u in range(U):
        jvu = jv + u
        v = plsc.load_gather(svbuf, [riota, jvu])
        f = plsc.load_gather(fbuf, [riota, jvu])
        s = v / (cv * f)
        plsc.store_scatter(svbuf, [riota, jvu], s)
        out.append(sums[u] + jnp.exp(s))
      return (jv + U, tuple(out))

    _, sums = lax.fori_loop(0, C // U, body,
                            (jnp.zeros((16,), jnp.int32),
                             tuple(zero for _ in range(U))))
    se = tree(list(sums), jnp.add)
    labv = labbuf[pl.ds(g * GR, GR)]
    svl = plsc.load_gather(svbuf, [riota, labv])
    sebuf[pl.ds(g * GR, GR)] = se
    svlbuf[pl.ds(g * GR, GR)] = svl

  def start_out(g):
    return pltpu.async_copy(svbufs[g % 3],
                            out_hbm.at[pl.ds(base + g * GR, GR), :],
                            outsems[g % 3])

  in_d = {}
  fine_d = {}
  out_d = {}
  in_d[0] = start_in(0)
  in_d[0].wait()
  am = p1(0)
  fine_d[0] = start_fine(0)
  ams = {0: am}
  if NG > 1:
    in_d[1] = start_in(1)
  for g in range(NG):
    if g + 1 < NG:
      in_d[g + 1].wait()
      ams[g + 1] = p1(g + 1)
      fine_d[g + 1] = start_fine(g + 1)
    fine_d[g].wait()
    p2(g, ams[g])
    out_d[g] = start_out(g)
    if g + 2 < NG:
      out_d[g - 1].wait() if g >= 1 else None
      in_d[g + 2] = start_in(g + 2)
  for g in range(max(0, NG - 3), NG):
    if g in out_d:
      out_d[g].wait()
  pltpu.sync_copy(sebuf, se_hbm.at[pl.ds(base, RW)])
  pltpu.sync_copy(svlbuf, svl_hbm.at[pl.ds(base, RW)])


def _tc_loss_body(se_ref, svl_ref, fine_ref, out_ref):
  nll = jnp.sum(jnp.log(se_ref[...]) - svl_ref[...]) / B
  reg = jnp.sum(jnp.abs(fine_ref[...] - 1.0)) / (C * C)
  out_ref[...] = jnp.full((1, 1), nll + reg, jnp.float32)


def kernel(Simple_vector, label_list, coarse_scaling_vector, fine_scaling_matrix):
  sc = pl.kernel(
      _sc_body,
      out_type=(
          jax.ShapeDtypeStruct((B, C), jnp.float32),
          jax.ShapeDtypeStruct((B,), jnp.float32),
          jax.ShapeDtypeStruct((B,), jnp.float32),
      ),
      mesh=plsc.VectorSubcoreMesh(core_axis_name="c", subcore_axis_name="s"),
      compiler_params=pltpu.CompilerParams(use_tc_tiling_on_sc=False,
                                           needs_layout_passes=False),
      scratch_types=[
          [pltpu.VMEM((GR, C), jnp.float32) for _ in range(3)],
          [pltpu.VMEM((GR, C), jnp.float32) for _ in range(2)],
          [pltpu.VMEM((GR,), jnp.int32) for _ in range(2)],
          pltpu.VMEM((RW,), jnp.int32),
          pltpu.VMEM((C,), jnp.float32),
          pltpu.VMEM((RW,), jnp.float32),
          pltpu.VMEM((RW,), jnp.float32),
          [pltpu.SemaphoreType.DMA for _ in range(3)],
          [pltpu.SemaphoreType.DMA for _ in range(2)],
          [pltpu.SemaphoreType.DMA for _ in range(3)],
      ],
  )
  sv, se, svl = sc(Simple_vector, label_list,
                   coarse_scaling_vector, fine_scaling_matrix)

  loss2 = pl.pallas_call(
      _tc_loss_body,
      out_shape=jax.ShapeDtypeStruct((1, 1), jnp.float32),
  )(se.reshape(32, 128), svl.reshape(32, 128), fine_scaling_matrix)
  loss = loss2[0, 0]
  return (sv, loss, jnp.zeros((), jnp.float32))


# speedup vs baseline: 1.3324x; 1.0002x over previous
import jax
import jax.numpy as jnp
from jax import lax
from jax.experimental import pallas as pl
from jax.experimental.pallas import tpu as pltpu
from jax.experimental.pallas import tpu_sc as plsc

C = 1000
B = 4096
NC = 2
NS = 16
NW = NC * NS
RW = B // NW
GR = 16
NG = RW // GR
U = 8


def _sc_body(sv_hbm, lab_hbm, coarse_hbm, fine_hbm,
             out_hbm, se_hbm, svl_hbm,
             svbufs, fbufs, idxbufs, labbuf, coarsebuf,
             sebuf, svlbuf, insems, fsems, outsems):
  cid = lax.axis_index("c")
  sid = lax.axis_index("s")
  wid = sid * NC + cid
  base = wid * RW
  riota = lax.iota(jnp.int32, 16)
  zero = jnp.zeros((16,), jnp.float32)
  neg_inf = jnp.full((16,), -jnp.inf, jnp.float32)

  pltpu.sync_copy(coarse_hbm, coarsebuf)
  pltpu.sync_copy(lab_hbm.at[pl.ds(base, RW)], labbuf)

  def merge_am(a, b):
    va, ia = a
    vb, ib = b
    p = (va > vb) | ((va == vb) & (ia < ib))
    return jnp.where(p, va, vb), jnp.where(p, ia, ib)

  def tree(xs, f):
    while len(xs) > 1:
      xs = [f(xs[i], xs[i + 1]) for i in range(0, len(xs), 2)]
    return xs[0]

  def start_in(g):
    return pltpu.async_copy(
        sv_hbm.at[pl.ds(base + g * GR, GR), :], svbufs[g % 3],
        insems[g % 3])

  def p1(g):
    svbuf = svbufs[g % 3]

    def body(jb, carry):
      jv = carry[0]
      accs = list(carry[1])
      out = []
      for u in range(U):
        jvu = jv + u
        v = plsc.load_gather(svbuf, [riota, jvu])
        mx, am = accs[u]
        p = v > mx
        out.append((jnp.where(p, v, mx), jnp.where(p, jvu, am)))
      return (jv + U, tuple(out))

    acc0 = tuple((neg_inf, jnp.zeros((16,), jnp.int32)) for _ in range(U))
    _, accs = lax.fori_loop(0, C // U, body,
                            (jnp.zeros((16,), jnp.int32), acc0))
    _, am = tree(list(accs), merge_am)
    idxbufs[g % 2][...] = am
    return am

  def start_fine(g):
    return pltpu.async_copy(fine_hbm.at[idxbufs[g % 2]], fbufs[g % 2],
                            fsems[g % 2])

  def p2(g, am):
    svbuf = svbufs[g % 3]
    fbuf = fbufs[g % 2]
    cv = plsc.load_gather(coarsebuf, [am])

    def body(jb, carry):
      jv = carry[0]
      sums = list(carry[1])
      out = []
      for u in range(U):
        jvu = jv + u
        v = plsc.load_gather(svbuf, [riota, jvu])
        f = plsc.load_gather(fbuf, [riota, jvu])
        s = v / (cv * f)
        plsc.store_scatter(svbuf, [riota, jvu], s)
        out.append(sums[u] + jnp.exp(s))
      return (jv + U, tuple(out))

    _, sums = lax.fori_loop(0, C // U, body,
                            (jnp.zeros((16,), jnp.int32),
                             tuple(zero for _ in range(U))))
    se = tree(list(sums), jnp.add)
    labv = labbuf[pl.ds(g * GR, GR)]
    svl = plsc.load_gather(svbuf, [riota, labv])
    sebuf[pl.ds(g * GR, GR)] = se
    svlbuf[pl.ds(g * GR, GR)] = svl

  def start_out(g):
    return pltpu.async_copy(svbufs[g % 3],
                            out_hbm.at[pl.ds(base + g * GR, GR), :],
                            outsems[g % 3])

  in_d = {}
  fine_d = {}
  out_d = {}
  in_d[0] = start_in(0)
  in_d[0].wait()
  am = p1(0)
  fine_d[0] = start_fine(0)
  ams = {0: am}
  if NG > 1:
    in_d[1] = start_in(1)
  for g in range(NG):
    if g + 1 < NG:
      with jax.named_scope("wait_in"):
        in_d[g + 1].wait()
      with jax.named_scope("p1"):
        ams[g + 1] = p1(g + 1)
      fine_d[g + 1] = start_fine(g + 1)
    with jax.named_scope("wait_fine"):
      fine_d[g].wait()
    with jax.named_scope("p2"):
      p2(g, ams[g])
    out_d[g] = start_out(g)
    if g + 2 < NG:
      out_d[g - 1].wait() if g >= 1 else None
      in_d[g + 2] = start_in(g + 2)
  for g in range(max(0, NG - 3), NG):
    if g in out_d:
      out_d[g].wait()
  pltpu.sync_copy(sebuf, se_hbm.at[pl.ds(base, RW)])
  pltpu.sync_copy(svlbuf, svl_hbm.at[pl.ds(base, RW)])


def _tc_loss_body(se_ref, svl_ref, fine_ref, out_ref):
  nll = jnp.sum(jnp.log(se_ref[...]) - svl_ref[...]) / B
  reg = jnp.sum(jnp.abs(fine_ref[...] - 1.0)) / (C * C)
  out_ref[...] = jnp.full((1, 1), nll + reg, jnp.float32)


def kernel(Simple_vector, label_list, coarse_scaling_vector, fine_scaling_matrix):
  sc = pl.kernel(
      _sc_body,
      out_type=(
          jax.ShapeDtypeStruct((B, C), jnp.float32),
          jax.ShapeDtypeStruct((B,), jnp.float32),
          jax.ShapeDtypeStruct((B,), jnp.float32),
      ),
      mesh=plsc.VectorSubcoreMesh(core_axis_name="c", subcore_axis_name="s"),
      compiler_params=pltpu.CompilerParams(use_tc_tiling_on_sc=False,
                                           needs_layout_passes=False),
      scratch_types=[
          [pltpu.VMEM((GR, C), jnp.float32) for _ in range(3)],
          [pltpu.VMEM((GR, C), jnp.float32) for _ in range(2)],
          [pltpu.VMEM((GR,), jnp.int32) for _ in range(2)],
          pltpu.VMEM((RW,), jnp.int32),
          pltpu.VMEM((C,), jnp.float32),
          pltpu.VMEM((RW,), jnp.float32),
          pltpu.VMEM((RW,), jnp.float32),
          [pltpu.SemaphoreType.DMA for _ in range(3)],
          [pltpu.SemaphoreType.DMA for _ in range(2)],
          [pltpu.SemaphoreType.DMA for _ in range(3)],
      ],
  )
  sv, se, svl = sc(Simple_vector, label_list,
                   coarse_scaling_vector, fine_scaling_matrix)

  loss2 = pl.pallas_call(
      _tc_loss_body,
      out_shape=jax.ShapeDtypeStruct((1, 1), jnp.float32),
  )(se.reshape(32, 128), svl.reshape(32, 128), fine_scaling_matrix)
  loss = loss2[0, 0]
  return (sv, loss, jnp.zeros((), jnp.float32))


# speedup vs baseline: 1.7867x; 1.3410x over previous
import jax
import jax.numpy as jnp
from jax import lax
from jax.experimental import pallas as pl
from jax.experimental.pallas import tpu as pltpu
from jax.experimental.pallas import tpu_sc as plsc

C = 1000
B = 4096
NC = 2
NS = 16
NW = NC * NS
RW = B // NW
GR = 16
NG = RW // GR
BR = 512


def _tc_argmax_body(sv_ref, idx_ref):
  x = sv_ref[...]
  m = jnp.max(x, axis=1, keepdims=True)
  ji = lax.broadcasted_iota(jnp.int32, (BR, C), 1)
  idx_ref[...] = jnp.min(jnp.where(x == m, ji, C), axis=1, keepdims=True)


def _sc_gather_body(idx_hbm, coarse_hbm, fine_hbm, g_hbm,
                    idxv, coarsebuf, fbufs, gsems, osems):
  cid = lax.axis_index("c")
  sid = lax.axis_index("s")
  wid = sid * NC + cid
  base = wid * RW
  riota = lax.iota(jnp.int32, 16)

  pltpu.sync_copy(coarse_hbm, coarsebuf)
  pltpu.sync_copy(idx_hbm.at[pl.ds(base, RW)], idxv)

  def start_gather(g):
    return pltpu.async_copy(fine_hbm.at[idxv.at[pl.ds(g * GR, GR)]],
                            fbufs[g % 3], gsems[g % 3])

  def start_out(g):
    return pltpu.async_copy(fbufs[g % 3],
                            g_hbm.at[pl.ds(base + g * GR, GR), :],
                            osems[g % 3])

  def mul_pass(g):
    fbuf = fbufs[g % 3]
    am = idxv[pl.ds(g * GR, GR)]
    cv = plsc.load_gather(coarsebuf, [am])

    def body(jb, jv):
      for u in range(8):
        jvu = jv + u
        f = plsc.load_gather(fbuf, [riota, jvu])
        plsc.store_scatter(fbuf, [riota, jvu], cv * f)
      return jv + 8

    lax.fori_loop(0, C // 8, body, jnp.zeros((16,), jnp.int32))

  gd = {0: start_gather(0)}
  if NG > 1:
    gd[1] = start_gather(1)
  od = {}
  for g in range(NG):
    gd[g].wait()
    mul_pass(g)
    od[g] = start_out(g)
    if g + 2 < NG:
      if g >= 1:
        od[g - 1].wait()
      gd[g + 2] = start_gather(g + 2)
  for g in range(max(0, NG - 3), NG):
    od[g].wait()


def _tc_scale_body(sv_ref, g_ref, lab_ref, out_ref, se_ref, svl_ref):
  x = sv_ref[...]
  s = x / g_ref[...]
  out_ref[...] = s
  se_ref[...] = jnp.sum(jnp.exp(s), axis=1, keepdims=True)
  ji = lax.broadcasted_iota(jnp.int32, (BR, C), 1)
  svl_ref[...] = jnp.sum(jnp.where(ji == lab_ref[...], s, 0.0),
                         axis=1, keepdims=True)


def _tc_loss_body(se_ref, svl_ref, fine_ref, out_ref):
  nll = jnp.sum(jnp.log(se_ref[...]) - svl_ref[...]) / B
  reg = jnp.sum(jnp.abs(fine_ref[...] - 1.0)) / (C * C)
  out_ref[...] = jnp.full((1, 1), nll + reg, jnp.float32)


def kernel(Simple_vector, label_list, coarse_scaling_vector, fine_scaling_matrix):
  nblk = B // BR
  idx2 = pl.pallas_call(
      _tc_argmax_body,
      grid=(nblk,),
      in_specs=[pl.BlockSpec((BR, C), lambda i: (i, 0))],
      out_specs=pl.BlockSpec((BR, 1), lambda i: (i, 0)),
      out_shape=jax.ShapeDtypeStruct((B, 1), jnp.int32),
  )(Simple_vector)
  idx = idx2.reshape(B)

  sc = pl.kernel(
      _sc_gather_body,
      out_type=jax.ShapeDtypeStruct((B, C), jnp.float32),
      mesh=plsc.VectorSubcoreMesh(core_axis_name="c", subcore_axis_name="s"),
      compiler_params=pltpu.CompilerParams(use_tc_tiling_on_sc=False,
                                           needs_layout_passes=False),
      scratch_types=[
          pltpu.VMEM((RW,), jnp.int32),
          pltpu.VMEM((C,), jnp.float32),
          [pltpu.VMEM((GR, C), jnp.float32) for _ in range(3)],
          [pltpu.SemaphoreType.DMA for _ in range(3)],
          [pltpu.SemaphoreType.DMA for _ in range(3)],
      ],
  )
  G = sc(idx, coarse_scaling_vector, fine_scaling_matrix)

  sv, se2, svl2 = pl.pallas_call(
      _tc_scale_body,
      grid=(nblk,),
      in_specs=[pl.BlockSpec((BR, C), lambda i: (i, 0)),
                pl.BlockSpec((BR, C), lambda i: (i, 0)),
                pl.BlockSpec((BR, 1), lambda i: (i, 0))],
      out_specs=[pl.BlockSpec((BR, C), lambda i: (i, 0)),
                 pl.BlockSpec((BR, 1), lambda i: (i, 0)),
                 pl.BlockSpec((BR, 1), lambda i: (i, 0))],
      out_shape=[jax.ShapeDtypeStruct((B, C), jnp.float32),
                 jax.ShapeDtypeStruct((B, 1), jnp.float32),
                 jax.ShapeDtypeStruct((B, 1), jnp.float32)],
  )(Simple_vector, G, label_list.reshape(B, 1))

  loss2 = pl.pallas_call(
      _tc_loss_body,
      out_shape=jax.ShapeDtypeStruct((1, 1), jnp.float32),
  )(se2, svl2, fine_scaling_matrix)
  loss = loss2[0, 0]
  return (sv, loss, jnp.zeros((), jnp.float32))


# speedup vs baseline: 2.3030x; 1.2890x over previous
import jax
import jax.numpy as jnp
from jax import lax
from jax.experimental import pallas as pl
from jax.experimental.pallas import tpu as pltpu
from jax.experimental.pallas import tpu_sc as plsc

C = 1000
B = 4096
NC = 2
NS = 16
NW = NC * NS
RW = B // NW
GR = 16
NG = RW // GR
BR = 512


def _tc_argmax_body(sv_ref, idx_ref):
  x = sv_ref[...]
  m = jnp.max(x, axis=1, keepdims=True)
  ji = lax.broadcasted_iota(jnp.int32, (BR, C), 1)
  idx_ref[...] = jnp.min(jnp.where(x == m, ji, C), axis=1, keepdims=True)


def _sc_gather_body(idx_hbm, coarse_hbm, fine_hbm, g_hbm, cv_hbm,
                    idxv, coarsebuf, cvbuf, fbufs, gsems, osems):
  cid = lax.axis_index("c")
  sid = lax.axis_index("s")
  wid = sid * NC + cid
  base = wid * RW

  pltpu.sync_copy(coarse_hbm, coarsebuf)
  pltpu.sync_copy(idx_hbm.at[pl.ds(base, RW)], idxv)

  def start_gather(g):
    return pltpu.async_copy(fine_hbm.at[idxv.at[pl.ds(g * GR, GR)]],
                            fbufs[g % 3], gsems[g % 3])

  def start_out(g):
    return pltpu.async_copy(fbufs[g % 3],
                            g_hbm.at[pl.ds(base + g * GR, GR), :],
                            osems[g % 3])

  gd = {0: start_gather(0)}
  if NG > 1:
    gd[1] = start_gather(1)
  if NG > 2:
    gd[2] = start_gather(2)
  od = {}
  for g in range(NG):
    gd[g].wait()
    od[g] = start_out(g)
    if g + 3 < NG:
      od[g].wait()
      gd[g + 3] = start_gather(g + 3)
  for k in range(RW // GR):
    am = idxv[pl.ds(k * GR, GR)]
    cvbuf[pl.ds(k * GR, GR)] = plsc.load_gather(coarsebuf, [am])
  pltpu.sync_copy(cvbuf, cv_hbm.at[pl.ds(base, RW)])
  for g in range(max(0, NG - 3), NG):
    od[g].wait()


def _tc_scale_body(sv_ref, g_ref, cv_ref, lab_ref, out_ref, se_ref, svl_ref):
  x = sv_ref[...]
  s = x / (cv_ref[...] * g_ref[...])
  out_ref[...] = s
  se_ref[...] = jnp.sum(jnp.exp(s), axis=1, keepdims=True)
  ji = lax.broadcasted_iota(jnp.int32, (BR, C), 1)
  svl_ref[...] = jnp.sum(jnp.where(ji == lab_ref[...], s, 0.0),
                         axis=1, keepdims=True)


def _tc_loss_body(se_ref, svl_ref, fine_ref, out_ref):
  nll = jnp.sum(jnp.log(se_ref[...]) - svl_ref[...]) / B
  reg = jnp.sum(jnp.abs(fine_ref[...] - 1.0)) / (C * C)
  out_ref[...] = jnp.full((1, 1), nll + reg, jnp.float32)


def kernel(Simple_vector, label_list, coarse_scaling_vector, fine_scaling_matrix):
  nblk = B // BR
  idx2 = pl.pallas_call(
      _tc_argmax_body,
      grid=(nblk,),
      in_specs=[pl.BlockSpec((BR, C), lambda i: (i, 0))],
      out_specs=pl.BlockSpec((BR, 1), lambda i: (i, 0)),
      out_shape=jax.ShapeDtypeStruct((B, 1), jnp.int32),
  )(Simple_vector)
  idx = idx2.reshape(B)

  sc = pl.kernel(
      _sc_gather_body,
      out_type=(jax.ShapeDtypeStruct((B, C), jnp.float32),
                jax.ShapeDtypeStruct((B,), jnp.float32)),
      mesh=plsc.VectorSubcoreMesh(core_axis_name="c", subcore_axis_name="s"),
      compiler_params=pltpu.CompilerParams(use_tc_tiling_on_sc=False,
                                           needs_layout_passes=False),
      scratch_types=[
          pltpu.VMEM((RW,), jnp.int32),
          pltpu.VMEM((C,), jnp.float32),
          pltpu.VMEM((RW,), jnp.float32),
          [pltpu.VMEM((GR, C), jnp.float32) for _ in range(3)],
          [pltpu.SemaphoreType.DMA for _ in range(3)],
          [pltpu.SemaphoreType.DMA for _ in range(3)],
      ],
  )
  G, cvals = sc(idx, coarse_scaling_vector, fine_scaling_matrix)

  sv, se2, svl2 = pl.pallas_call(
      _tc_scale_body,
      grid=(nblk,),
      in_specs=[pl.BlockSpec((BR, C), lambda i: (i, 0)),
                pl.BlockSpec((BR, C), lambda i: (i, 0)),
                pl.BlockSpec((BR, 1), lambda i: (i, 0)),
                pl.BlockSpec((BR, 1), lambda i: (i, 0))],
      out_specs=[pl.BlockSpec((BR, C), lambda i: (i, 0)),
                 pl.BlockSpec((BR, 1), lambda i: (i, 0)),
                 pl.BlockSpec((BR, 1), lambda i: (i, 0))],
      out_shape=[jax.ShapeDtypeStruct((B, C), jnp.float32),
                 jax.ShapeDtypeStruct((B, 1), jnp.float32),
                 jax.ShapeDtypeStruct((B, 1), jnp.float32)],
  )(Simple_vector, G, cvals.reshape(B, 1), label_list.reshape(B, 1))

  loss2 = pl.pallas_call(
      _tc_loss_body,
      out_shape=jax.ShapeDtypeStruct((1, 1), jnp.float32),
  )(se2, svl2, fine_scaling_matrix)
  loss = loss2[0, 0]
  return (sv, loss, jnp.zeros((), jnp.float32))
